# baseline (speedup 1.0000x reference)
import functools

import jax
import jax.numpy as jnp
from jax import lax
from jax.experimental import pallas as pl
from jax.experimental.pallas import tpu as pltpu
from jax.experimental.pallas import tpu_sc as plsc

NC = 2
NS = 16
NW = NC * NS
L = 16

CHUNK = 128


def _sc_body(B, D, b_per_w,
             user_idx_hbm, item_idx_hbm, know_hbm,
             theta_tab_hbm, a_tab_hbm, b_tab_hbm,
             pred_out, theta_out, a_out, b_out,
             uidx_v, iidx_v, theta_v, a_v, b_v, know_v, pred_v,
             sem_g, sem_o):
    n_chunks = b_per_w // CHUNK
    wid = lax.axis_index("s") * NC + lax.axis_index("c")
    base = wid * b_per_w

    for j in range(n_chunks):
        pltpu.sync_copy(user_idx_hbm.at[pl.ds(base + j * CHUNK, CHUNK)],
                        uidx_v.at[j])
        pltpu.sync_copy(item_idx_hbm.at[pl.ds(base + j * CHUNK, CHUNK)],
                        iidx_v.at[j])

    copies = []
    for j in range(n_chunks):
        sl = pl.ds(j * CHUNK, CHUNK)
        copies.append(pltpu.async_copy(
            theta_tab_hbm.at[uidx_v.at[j]], theta_v.at[sl], sem_g))
        copies.append(pltpu.async_copy(
            a_tab_hbm.at[iidx_v.at[j]], a_v.at[sl], sem_g))
        copies.append(pltpu.async_copy(
            b_tab_hbm.at[iidx_v.at[j]], b_v.at[sl], sem_g))
    copies.append(pltpu.async_copy(
        know_hbm.at[pl.ds(base, b_per_w)], know_v, sem_g))
    for c in copies:
        c.wait()

    out_copies = [
        pltpu.async_copy(theta_v, theta_out.at[pl.ds(base, b_per_w)], sem_o),
        pltpu.async_copy(a_v, a_out.at[pl.ds(base, b_per_w)], sem_o),
        pltpu.async_copy(b_v, b_out.at[pl.ds(base, b_per_w)], sem_o),
    ]

    n_sub = D // L
    lane = lax.iota(jnp.int32, L)

    def group_body(g, carry):
        base_r = g * L
        zvec = jnp.zeros((L,), jnp.float32)
        for rl in range(L):
            r = base_r + rl
            acc = (theta_v[r, pl.ds(0, L)] * a_v[r, pl.ds(0, L)]
                   * know_v[r, pl.ds(0, L)])
            for c in range(1, n_sub):
                sl = pl.ds(c * L, L)
                acc = acc + theta_v[r, sl] * a_v[r, sl] * know_v[r, sl]
            zvec = jnp.where(lane == rl, jnp.sum(acc), zvec)
        z = zvec - b_v[pl.ds(base_r, L)]
        pred_v[pl.ds(base_r, L)] = 1.0 / (1.0 + jnp.exp(-z))
        return carry

    lax.fori_loop(0, b_per_w // L, group_body, 0)

    out_copies.append(pltpu.async_copy(
        pred_v, pred_out.at[pl.ds(base, b_per_w)], sem_o))
    for c in out_copies:
        c.wait()


@functools.partial(jax.jit, static_argnames=())
def _emb_icd(user_idx, item_idx, know, theta_table, a_table, b_table):
    B, D = know.shape
    assert B % (NW * CHUNK) == 0 and D % L == 0
    b_per_w = B // NW
    n_chunks = b_per_w // CHUNK

    mesh = plsc.VectorSubcoreMesh(core_axis_name="c", subcore_axis_name="s",
                                  num_cores=NC, num_subcores=NS)
    fn = pl.kernel(
        functools.partial(_sc_body, B, D, b_per_w),
        out_type=(
            jax.ShapeDtypeStruct((B,), jnp.float32),
            jax.ShapeDtypeStruct((B, D), jnp.float32),
            jax.ShapeDtypeStruct((B, D), jnp.float32),
            jax.ShapeDtypeStruct((B,), jnp.float32),
        ),
        mesh=mesh,
        scratch_types=[
            pltpu.VMEM((n_chunks, CHUNK), jnp.int32),
            pltpu.VMEM((n_chunks, CHUNK), jnp.int32),
            pltpu.VMEM((b_per_w, D), jnp.float32),
            pltpu.VMEM((b_per_w, D), jnp.float32),
            pltpu.VMEM((b_per_w,), jnp.float32),
            pltpu.VMEM((b_per_w, D), jnp.float32),
            pltpu.VMEM((b_per_w,), jnp.float32),
            pltpu.SemaphoreType.DMA,
            pltpu.SemaphoreType.DMA,
        ],
        compiler_params=pltpu.CompilerParams(needs_layout_passes=False,
                                             use_tc_tiling_on_sc=False),
        name="emb_icd_sc",
    )
    return fn(user_idx, item_idx, know, theta_table, a_table,
              b_table.reshape(-1))


def kernel(user_idx, item_idx, know, theta_table, a_table, b_table):
    user_idx = user_idx.astype(jnp.int32)
    item_idx = item_idx.astype(jnp.int32)
    pred, theta, a, b_flat = _emb_icd(user_idx, item_idx, know,
                                      theta_table, a_table, b_table)
    return (pred, theta, a, b_flat.reshape(-1, 1))
